# Initial kernel scaffold; baseline (speedup 1.0000x reference)
import jax
import jax.numpy as jnp
from jax.experimental import pallas as pl


def kernel(tokens, kernel):
    raise NotImplementedError("write your pallas kernel here")


# speedup vs baseline: 1.5522x; 1.5522x over previous
import functools

import jax
import jax.numpy as jnp
from jax import lax
from jax.experimental import pallas as pl
from jax.experimental.pallas import tpu as pltpu
from jax.experimental.pallas import tpu_sc as plsc

D = 512
NC = 2
NS = 16
NW = NC * NS
C = 64
NCHUNK = 100


def _emb_body(table_hbm, idx_hbm, out_hbm, idx_v, buf0, buf1,
              gsem0, gsem1, ssem0, ssem1):
  wid = lax.axis_index("s") * NC + lax.axis_index("c")
  base = wid * (NCHUNK * C)

  pltpu.sync_copy(idx_hbm.at[wid], idx_v)

  bufs = (buf0, buf1)
  gsems = (gsem0, gsem1)
  ssems = (ssem0, ssem1)

  def start_gather(j, b):
    pltpu.async_copy(table_hbm.at[idx_v.at[j]], bufs[b], gsems[b])

  def wait_gather(j, b):
    pltpu.make_async_copy(table_hbm.at[idx_v.at[j]], bufs[b], gsems[b]).wait()

  def out_slice(j):
    return out_hbm.at[pl.ds(base + j * C, C)]

  def start_scatter(j, b):
    pltpu.async_copy(bufs[b], out_slice(j), ssems[b])

  def wait_scatter(j, b):
    pltpu.make_async_copy(bufs[b], out_slice(j), ssems[b]).wait()

  start_gather(0, 0)
  start_gather(1, 1)

  def body(i, carry):
    j0 = i * 2
    for b in range(2):
      j = j0 + b
      wait_gather(j, b)
      start_scatter(j, b)
    for b in range(2):
      j = j0 + b

      @pl.when(j + 2 < NCHUNK)
      def _():
        wait_scatter(j, b)
        start_gather(j + 2, b)

    return carry

  lax.fori_loop(0, NCHUNK // 2, body, 0)

  wait_scatter(NCHUNK - 2, 0)
  wait_scatter(NCHUNK - 1, 1)


@functools.partial(jax.jit, static_argnames=())
def kernel(tokens, kernel):
  table = kernel
  b, t = tokens.shape
  idx3 = tokens.reshape(NW, NCHUNK, C).astype(jnp.int32)

  mesh = plsc.VectorSubcoreMesh(core_axis_name="c", subcore_axis_name="s")
  emb = pl.kernel(
      _emb_body,
      mesh=mesh,
      out_type=jax.ShapeDtypeStruct((NW * NCHUNK * C, D), jnp.float32),
      scratch_types=[
          pltpu.VMEM((NCHUNK, C), jnp.int32),
          pltpu.VMEM((C, D), jnp.float32),
          pltpu.VMEM((C, D), jnp.float32),
          pltpu.SemaphoreType.DMA,
          pltpu.SemaphoreType.DMA,
          pltpu.SemaphoreType.DMA,
          pltpu.SemaphoreType.DMA,
      ],
  )
  out = emb(table, idx3)
  return out.reshape(b, t, D)


# speedup vs baseline: 1.5576x; 1.0035x over previous
import functools

import jax
import jax.numpy as jnp
from jax import lax
from jax.experimental import pallas as pl
from jax.experimental.pallas import tpu as pltpu
from jax.experimental.pallas import tpu_sc as plsc

D = 512
NC = 2
NS = 16
NW = NC * NS
N = 4096 * 50
PER_W = N // NW
C = 40
NCHUNK = PER_W // C
NBUF = 4


def _emb_body(table_hbm, idx_hbm, out_hbm, idx_v,
              buf0, buf1, buf2, buf3,
              gsem0, gsem1, gsem2, gsem3,
              ssem0, ssem1, ssem2, ssem3):
  wid = lax.axis_index("s") * NC + lax.axis_index("c")
  base = wid * PER_W

  pltpu.sync_copy(idx_hbm.at[pl.ds(base, PER_W)], idx_v)

  bufs = (buf0, buf1, buf2, buf3)
  gsems = (gsem0, gsem1, gsem2, gsem3)
  ssems = (ssem0, ssem1, ssem2, ssem3)

  def start_gather(j, b):
    pltpu.async_copy(
        table_hbm.at[idx_v.at[pl.ds(j * C, C)]], bufs[b], gsems[b])

  def wait_gather(j, b):
    pltpu.make_async_copy(
        table_hbm.at[idx_v.at[pl.ds(j * C, C)]], bufs[b], gsems[b]).wait()

  def start_scatter(j, b):
    pltpu.async_copy(bufs[b], out_hbm.at[pl.ds(base + j * C, C)], ssems[b])

  def wait_scatter(j, b):
    pltpu.make_async_copy(
        bufs[b], out_hbm.at[pl.ds(base + j * C, C)], ssems[b]).wait()

  for b in range(NBUF):
    start_gather(b, b)

  def body(i, carry):
    j0 = i * NBUF
    for b in range(NBUF):
      j = j0 + b
      wait_gather(j, b)
      start_scatter(j, b)
    for b in range(NBUF):
      j = j0 + b

      @pl.when(j + NBUF < NCHUNK)
      def _():
        wait_scatter(j, b)
        start_gather(j + NBUF, b)

    return carry

  lax.fori_loop(0, NCHUNK // NBUF, body, 0)

  for b in range(NBUF):
    wait_scatter(NCHUNK - NBUF + b, b)


@functools.partial(jax.jit, static_argnames=())
def kernel(tokens, kernel):
  table = kernel
  b, t = tokens.shape
  idx = tokens.astype(jnp.int32).reshape(-1)

  mesh = plsc.VectorSubcoreMesh(core_axis_name="c", subcore_axis_name="s")
  emb = pl.kernel(
      _emb_body,
      mesh=mesh,
      out_type=jax.ShapeDtypeStruct((N, D), jnp.float32),
      scratch_types=(
          [pltpu.VMEM((PER_W,), jnp.int32)]
          + [pltpu.VMEM((C, D), jnp.float32) for _ in range(NBUF)]
          + [pltpu.SemaphoreType.DMA for _ in range(2 * NBUF)]
      ),
  )
  return emb(table, idx).reshape(b, t, D)
